# baseline (speedup 1.0000x reference)
import functools

import jax
import jax.numpy as jnp
from jax import lax
from jax.experimental import pallas as pl
from jax.experimental.pallas import tpu as pltpu
from jax.experimental.pallas import tpu_sc as plsc


def _sc_dots(v_i, v_j, negsamples, second, context):
    B = v_i.shape[0]
    K = negsamples.shape[0]
    D = second.shape[1]
    info = plsc.get_sparse_core_info()
    NC, NS, L = 1, info.num_subcores, info.num_lanes
    NW = NC * NS
    BW = B // NW
    NCH = D // L

    mesh = plsc.VectorSubcoreMesh(core_axis_name="c", subcore_axis_name="s",
                                  num_cores=NC)

    @functools.partial(
        pl.kernel,
        mesh=mesh,
        out_type=jax.ShapeDtypeStruct(((1 + K) * B, L), jnp.float32),
        compiler_params=pltpu.CompilerParams(use_tc_tiling_on_sc=False),
        scratch_types=[
            pltpu.VMEM((BW,), jnp.int32),
            pltpu.VMEM((BW,), jnp.int32),
            pltpu.VMEM((K, BW), jnp.int32),
            pltpu.VMEM((BW, D), jnp.float32),
            pltpu.VMEM((BW, D), jnp.float32),
            pltpu.VMEM((2, BW, D), jnp.float32),
            pltpu.VMEM((1 + K, BW, L), jnp.float32),
            pltpu.SemaphoreType.DMA,
        ],
    )
    def k(vi_hbm, vj_hbm, neg_hbm, second_hbm, context_hbm, out_hbm,
          vi_idx, vj_idx, neg_idx, vi_rows, vj_rows, neg_rows, out_v, sem):
        wid = lax.axis_index("s") * NC + lax.axis_index("c")
        base = wid * BW

        pltpu.sync_copy(vi_hbm.at[pl.ds(base, BW)], vi_idx)
        pltpu.sync_copy(vj_hbm.at[pl.ds(base, BW)], vj_idx)
        for kk in range(K):
            pltpu.sync_copy(neg_hbm.at[pl.ds(kk * B + base, BW)],
                            neg_idx.at[kk])

        cp_vi = pltpu.async_copy(second_hbm.at[vi_idx], vi_rows, sem)
        cp_vj = pltpu.async_copy(context_hbm.at[vj_idx], vj_rows, sem)
        cp_neg = [pltpu.async_copy(context_hbm.at[neg_idx.at[0]],
                                   neg_rows.at[0], sem)]
        cp_vi.wait()
        cp_vj.wait()

        @plsc.parallel_loop(0, BW, unroll=8)
        def pos_body(g):
            acc = vi_rows[g, pl.ds(0, L)] * vj_rows[g, pl.ds(0, L)]
            for c in range(1, NCH):
                acc = acc + (vi_rows[g, pl.ds(c * L, L)]
                             * vj_rows[g, pl.ds(c * L, L)])
            out_v[0, g, :] = acc

        for kk in range(K):
            if kk + 1 < K:
                cp_neg.append(
                    pltpu.async_copy(context_hbm.at[neg_idx.at[kk + 1]],
                                     neg_rows.at[(kk + 1) % 2], sem))
            cp_neg[kk].wait()

            @plsc.parallel_loop(0, BW, unroll=8)
            def neg_body(g, _kk=kk):
                buf = _kk % 2
                acc = vi_rows[g, pl.ds(0, L)] * neg_rows[buf, g, pl.ds(0, L)]
                for c in range(1, NCH):
                    acc = acc + (vi_rows[g, pl.ds(c * L, L)]
                                 * neg_rows[buf, g, pl.ds(c * L, L)])
                out_v[1 + _kk, g, :] = acc

        for d in range(1 + K):
            pltpu.sync_copy(out_v.at[d], out_hbm.at[pl.ds(d * B + base, BW)])

    return k(v_i, v_j, negsamples.reshape(K * B), second, context)


def _tc_finalize(x, batch, num_dots, lanes):
    R, C = x.shape
    G = C // lanes
    RD = batch // G

    def body(x_ref, o_ref):
        xs = x_ref[...]
        col = lax.broadcasted_iota(jnp.int32, (C, G), 0)
        grp = lax.broadcasted_iota(jnp.int32, (C, G), 1)
        a = (col // lanes == grp).astype(jnp.float32)
        s = jnp.dot(xs, a, preferred_element_type=jnp.float32)

        def logsig(v):
            return jnp.minimum(v, 0.0) - jnp.log1p(jnp.exp(-jnp.abs(v)))

        acc = logsig(s[0:RD])
        for d in range(1, num_dots):
            acc = acc + logsig(-s[d * RD:(d + 1) * RD])
        o_ref[...] = jnp.broadcast_to(-(jnp.sum(acc) / batch), (1, 1))

    return pl.pallas_call(
        body,
        out_shape=jax.ShapeDtypeStruct((1, 1), jnp.float32),
    )(x)


def kernel(nodeindex, v_i, v_j, negsamples, first_embeddings,
           second_embeddings, context_embeddings):
    del nodeindex, first_embeddings
    B = v_i.shape[0]
    K = negsamples.shape[0]
    L = 16
    dots = _sc_dots(v_i, v_j, negsamples, second_embeddings,
                    context_embeddings)
    x = dots.reshape(((1 + K) * B * L) // 128, 128)
    loss = _tc_finalize(x, B, 1 + K, L)
    return loss[0, 0]


# speedup vs baseline: 1.1343x; 1.1343x over previous
import functools

import jax
import jax.numpy as jnp
from jax import lax
from jax.experimental import pallas as pl
from jax.experimental.pallas import tpu as pltpu
from jax.experimental.pallas import tpu_sc as plsc


def _sc_dots(idx_packed, second, context, NW, BW, K, L):
    D = second.shape[1]
    NC = 2
    NCH = D // L
    NI = 2 + K
    ND = 1 + K

    mesh = plsc.VectorSubcoreMesh(core_axis_name="c", subcore_axis_name="s",
                                  num_cores=NC)

    @functools.partial(
        pl.kernel,
        mesh=mesh,
        out_type=jax.ShapeDtypeStruct((NW * ND, BW, L), jnp.float32),
        compiler_params=pltpu.CompilerParams(use_tc_tiling_on_sc=False),
        scratch_types=[
            pltpu.VMEM((NI, BW), jnp.int32),
            pltpu.VMEM((BW, D), jnp.float32),
            pltpu.VMEM((BW, D), jnp.float32),
            pltpu.VMEM((K, BW, D), jnp.float32),
            pltpu.VMEM((ND, BW, L), jnp.float32),
            pltpu.SemaphoreType.DMA,
        ],
    )
    def k(idx_hbm, second_hbm, context_hbm, out_hbm,
          idx_v, vi_rows, vj_rows, neg_rows, out_v, sem):
        wid = lax.axis_index("s") * NC + lax.axis_index("c")

        pltpu.sync_copy(idx_hbm.at[pl.ds(wid * NI, NI)], idx_v)

        cps = [
            pltpu.async_copy(second_hbm.at[idx_v.at[0]], vi_rows, sem),
            pltpu.async_copy(context_hbm.at[idx_v.at[1]], vj_rows, sem),
        ]
        for kk in range(K):
            cps.append(
                pltpu.async_copy(context_hbm.at[idx_v.at[2 + kk]],
                                 neg_rows.at[kk], sem))
        cps[0].wait()
        cps[1].wait()

        @plsc.parallel_loop(0, BW, unroll=8)
        def pos_body(g):
            acc = vi_rows[g, pl.ds(0, L)] * vj_rows[g, pl.ds(0, L)]
            for c in range(1, NCH):
                acc = acc + (vi_rows[g, pl.ds(c * L, L)]
                             * vj_rows[g, pl.ds(c * L, L)])
            out_v[0, g, :] = acc

        for kk in range(K):
            cps[2 + kk].wait()

            @plsc.parallel_loop(0, BW, unroll=8)
            def neg_body(g, _kk=kk):
                acc = (vi_rows[g, pl.ds(0, L)]
                       * neg_rows[_kk, g, pl.ds(0, L)])
                for c in range(1, NCH):
                    acc = acc + (vi_rows[g, pl.ds(c * L, L)]
                                 * neg_rows[_kk, g, pl.ds(c * L, L)])
                out_v[1 + _kk, g, :] = acc

        pltpu.sync_copy(out_v, out_hbm.at[pl.ds(wid * ND, ND)])

    return k(idx_packed, second, context)


def _tc_finalize(x, batch, num_dots, block_rows):
    R, C = x.shape
    L = 16
    G = C // L

    def body(x_ref, o_ref):
        xs = x_ref[...]
        col = lax.broadcasted_iota(jnp.int32, (C, G), 0)
        grp = lax.broadcasted_iota(jnp.int32, (C, G), 1)
        a = (col // L == grp).astype(jnp.float32)
        s = jnp.dot(xs, a, preferred_element_type=jnp.float32)

        row = lax.broadcasted_iota(jnp.int32, (R, G), 0)
        d = (row // block_rows) % num_dots
        v = jnp.where(d == 0, s, -s)
        acc = jnp.minimum(v, 0.0) - jnp.log1p(jnp.exp(-jnp.abs(v)))
        o_ref[...] = jnp.broadcast_to(-(jnp.sum(acc) / batch), (1, 1))

    return pl.pallas_call(
        body,
        out_shape=jax.ShapeDtypeStruct((1, 1), jnp.float32),
    )(x)


def kernel(nodeindex, v_i, v_j, negsamples, first_embeddings,
           second_embeddings, context_embeddings):
    del nodeindex, first_embeddings
    B = v_i.shape[0]
    K = negsamples.shape[0]
    L = 16
    NW = 32
    BW = B // NW

    idx = jnp.concatenate(
        [v_i.reshape(1, B), v_j.reshape(1, B), negsamples], axis=0)
    idx_packed = (idx.reshape(2 + K, NW, BW)
                  .transpose(1, 0, 2)
                  .reshape(NW * (2 + K), BW))

    dots = _sc_dots(idx_packed, second_embeddings, context_embeddings,
                    NW, BW, K, L)
    x = dots.reshape((NW * (1 + K) * BW * L) // 128, 128)
    block_rows = (BW * L) // 128
    loss = _tc_finalize(x, B, 1 + K, block_rows)
    return loss[0, 0]
